# Initial kernel scaffold; baseline (speedup 1.0000x reference)
import jax
import jax.numpy as jnp
from jax.experimental import pallas as pl


def kernel(inputs, targets):
    raise NotImplementedError("write your pallas kernel here")


# speedup vs baseline: 7.1991x; 7.1991x over previous
import functools

import jax
import jax.numpy as jnp
from jax.experimental import pallas as pl
from jax.experimental.pallas import tpu as pltpu

_B, _C, _H, _W = 4, 19, 512, 512
_HW = _H * _W
_LANES = 128
_ROWS = _HW // _LANES
_RA = 64
_RB = 512
_NHI = 64
_NLO = 64
_NBINS = _NHI * _NLO
_SCALE = _NBINS / 16.0
_K = (_B * _HW) // 2


def _ce_hist_kernel(x_ref, t_ref, ce_ref, cnt_ref, acc_ref):
    b = pl.program_id(0)
    j = pl.program_id(1)
    x = x_ref[0]
    t = t_ref[0]
    m = jnp.max(x, axis=0)
    s = jnp.sum(jnp.exp(x - m[None]), axis=0)
    lse = jnp.log(s) + m
    cls = jax.lax.broadcasted_iota(jnp.int32, (_C, _RA, _LANES), 0)
    xt = jnp.sum(jnp.where(cls == t[None], x, 0.0), axis=0)
    ce = lse - xt
    ce_ref[0] = ce

    p = _RA * _LANES
    cef = ce.reshape(1, p)
    idx = jnp.clip(jnp.floor(cef * _SCALE), 0.0, float(_NBINS - 1)).astype(jnp.int32)
    hi = idx >> 6
    lo = idx & (_NLO - 1)
    lane = jax.lax.broadcasted_iota(jnp.int32, (_NHI, p), 0)
    ohhi = (hi == lane).astype(jnp.bfloat16)
    ohlo = (lo == lane).astype(jnp.bfloat16)
    cnt = jax.lax.dot_general(ohhi, ohlo, (((1,), (1,)), ((), ())),
                              preferred_element_type=jnp.float32)

    @pl.when(jnp.logical_and(b == 0, j == 0))
    def _init():
        acc_ref[...] = cnt

    @pl.when(jnp.logical_not(jnp.logical_and(b == 0, j == 0)))
    def _acc():
        acc_ref[...] += cnt

    @pl.when(jnp.logical_and(b == _B - 1, j == pl.num_programs(1) - 1))
    def _emit():
        cnt_ref[...] = acc_ref[...]


def _select_kernel(cnt_ref, ce_ref, out_ref, s_ref):
    step = pl.program_id(0)

    cnt2d = cnt_ref[...]
    a = jax.lax.broadcasted_iota(jnp.int32, (_NHI, _NLO), 0)
    bcol = jax.lax.broadcasted_iota(jnp.int32, (_NHI, _NLO), 1)
    tri_ge = (a >= bcol).astype(jnp.bfloat16)
    tri_gt = (bcol > a).astype(jnp.bfloat16)
    sufrow = jax.lax.dot_general(cnt2d.astype(jnp.bfloat16), tri_ge,
                                 (((1,), (0,)), ((), ())),
                                 preferred_element_type=jnp.float32)
    rowtot = sufrow[:, 0:1]
    above = jax.lax.dot_general(tri_gt, rowtot.astype(jnp.bfloat16),
                                (((1,), (0,)), ((), ())),
                                preferred_element_type=jnp.float32)
    sufc = above + sufrow
    key = a * _NLO + bcol
    sel = jnp.where(sufc >= float(_K), key, -1)
    bstar = jnp.max(sel)
    th = bstar.astype(jnp.float32) * (1.0 / _SCALE)

    ce = ce_ref[...]
    mask = ce >= th
    part_sum = jnp.sum(jnp.where(mask, ce, 0.0))
    part_cnt = jnp.sum(mask.astype(jnp.float32))

    @pl.when(step == 0)
    def _init():
        s_ref[0] = part_sum
        s_ref[1] = part_cnt

    @pl.when(step != 0)
    def _acc():
        s_ref[0] += part_sum
        s_ref[1] += part_cnt

    @pl.when(step == pl.num_programs(0) - 1)
    def _emit():
        out_ref[...] = (s_ref[0] / s_ref[1]) * jnp.ones((1, 1), jnp.float32)


@functools.partial(jax.jit, static_argnames=())
def kernel(inputs, targets):
    x4 = inputs.reshape(_B, _C, _ROWS, _LANES)
    t3 = targets.astype(jnp.int32).reshape(_B, _ROWS, _LANES)

    nj = _ROWS // _RA
    ce, cnt2d = pl.pallas_call(
        _ce_hist_kernel,
        grid=(_B, nj),
        in_specs=[
            pl.BlockSpec((1, _C, _RA, _LANES), lambda b, j: (b, 0, j, 0)),
            pl.BlockSpec((1, _RA, _LANES), lambda b, j: (b, j, 0)),
        ],
        out_specs=[
            pl.BlockSpec((1, _RA, _LANES), lambda b, j: (b, j, 0)),
            pl.BlockSpec((_NHI, _NLO), lambda b, j: (0, 0)),
        ],
        out_shape=[
            jax.ShapeDtypeStruct((_B, _ROWS, _LANES), jnp.float32),
            jax.ShapeDtypeStruct((_NHI, _NLO), jnp.float32),
        ],
        scratch_shapes=[pltpu.VMEM((_NHI, _NLO), jnp.float32)],
        compiler_params=pltpu.CompilerParams(
            dimension_semantics=("arbitrary", "arbitrary"),
        ),
    )(x4, t3)

    ce2 = ce.reshape(_B * _ROWS, _LANES)
    nsteps = (_B * _ROWS) // _RB
    out = pl.pallas_call(
        _select_kernel,
        grid=(nsteps,),
        in_specs=[
            pl.BlockSpec((_NHI, _NLO), lambda i: (0, 0)),
            pl.BlockSpec((_RB, _LANES), lambda i: (i, 0)),
        ],
        out_specs=pl.BlockSpec((1, 1), lambda i: (0, 0)),
        out_shape=jax.ShapeDtypeStruct((1, 1), jnp.float32),
        scratch_shapes=[pltpu.SMEM((2,), jnp.float32)],
        compiler_params=pltpu.CompilerParams(
            dimension_semantics=("arbitrary",),
        ),
    )(cnt2d, ce2)
    return out[0, 0]


# speedup vs baseline: 8.4086x; 1.1680x over previous
import functools

import jax
import jax.numpy as jnp
from jax import lax
from jax.experimental import pallas as pl
from jax.experimental.pallas import tpu as pltpu
from jax.experimental.pallas import tpu_sc as plsc

_B, _C, _H, _W = 4, 19, 512, 512
_HW = _H * _W
_LANES = 128
_ROWS = _HW // _LANES
_RA = 2048
_N = _B * _HW
_K = _N // 2
_NB = 4096
_SCALE = _NB / 16.0
_NC, _NS, _L = 2, 16, 16
_NW = _NC * _NS
_CHUNK = _N // _NW
_VECS = _CHUNK // _L
_BINV = _NB // _L
_BPW = _NB // _NS

_sc_mesh = plsc.VectorSubcoreMesh(core_axis_name="c", subcore_axis_name="s",
                                  num_cores=_NC, num_subcores=_NS)
_sc_params = pltpu.CompilerParams(needs_layout_passes=False)


def _ce_kernel(x_ref, t_ref, ce_ref):
    x = x_ref[0]
    t = t_ref[0]
    m = jnp.max(x, axis=0)
    s = jnp.sum(jnp.exp(x - m[None]), axis=0)
    lse = jnp.log(s) + m
    cls = jax.lax.broadcasted_iota(jnp.int32, (_C, _RA, _LANES), 0)
    xt = jnp.sum(jnp.where(cls == t[None], x, 0.0), axis=0)
    ce_ref[0] = lse - xt


@functools.partial(
    pl.kernel, mesh=_sc_mesh,
    out_type=jax.ShapeDtypeStruct((_NC, _NB), jnp.float32),
    scratch_types=[
        pltpu.VMEM((_CHUNK,), jnp.float32),
        pltpu.VMEM((_NB,), jnp.float32),
        pltpu.VMEM((_BPW,), jnp.float32),
        pltpu.VMEM_SHARED((_NS, _NB), jnp.float32),
    ],
    compiler_params=_sc_params,
)
def _sc_hist(ce_hbm, hist_hbm, data_v, hist_v, merge_v, slots):
    c = lax.axis_index("c")
    s = lax.axis_index("s")
    wid = s * _NC + c
    zeros = jnp.zeros((_L,), jnp.float32)
    ones = jnp.ones((_L,), jnp.float32)

    pltpu.sync_copy(ce_hbm.at[pl.ds(wid * _CHUNK, _CHUNK)], data_v)

    def zb(i, _):
        hist_v[pl.ds(i * _L, _L)] = zeros
        return 0
    lax.fori_loop(0, _BINV, zb, 0)

    def hb(i, _):
        v = data_v[pl.ds(i * _L, _L)]
        idx = jnp.clip(v * _SCALE, 0.0, float(_NB - 1)).astype(jnp.int32)
        plsc.addupdate_scatter(hist_v, [idx], ones)
        return 0
    lax.fori_loop(0, _VECS, hb, 0)

    pltpu.sync_copy(hist_v, slots.at[s])
    plsc.subcore_barrier()

    def zm(i, _):
        merge_v[pl.ds(i * _L, _L)] = zeros
        return 0
    lax.fori_loop(0, _BPW // _L, zm, 0)

    def ms(w, _):
        pltpu.sync_copy(slots.at[w, pl.ds(s * _BPW, _BPW)],
                        hist_v.at[pl.ds(0, _BPW)])
        def mr(i, __):
            merge_v[pl.ds(i * _L, _L)] = (merge_v[pl.ds(i * _L, _L)]
                                          + hist_v[pl.ds(i * _L, _L)])
            return 0
        lax.fori_loop(0, _BPW // _L, mr, 0)
        return 0
    lax.fori_loop(0, _NS, ms, 0)

    pltpu.sync_copy(merge_v, hist_hbm.at[c, pl.ds(s * _BPW, _BPW)])


@functools.partial(
    pl.kernel, mesh=_sc_mesh,
    out_type=jax.ShapeDtypeStruct((_NC, _L), jnp.float32),
    scratch_types=[
        pltpu.VMEM((_CHUNK,), jnp.float32),
        pltpu.VMEM((_NC, _NB), jnp.float32),
        pltpu.VMEM((_L,), jnp.float32),
        pltpu.VMEM_SHARED((_NS, _L), jnp.float32),
    ],
    compiler_params=_sc_params,
)
def _sc_select(ce_hbm, hist_hbm, out_hbm, data_v, hist_v, stage_v, slots):
    c = lax.axis_index("c")
    s = lax.axis_index("s")
    wid = s * _NC + c
    lane = lax.iota(jnp.int32, _L)

    pltpu.sync_copy(ce_hbm.at[pl.ds(wid * _CHUNK, _CHUNK)], data_v)
    pltpu.sync_copy(hist_hbm, hist_v)

    def sb(j, carry):
        tot_above, bstar_f = carry
        b = _BINV - 1 - j
        v = hist_v[0, pl.ds(b * _L, _L)] + hist_v[1, pl.ds(b * _L, _L)]
        total = jnp.sum(v, axis=0)
        pre = plsc.cumsum(v)
        sfx = (total + tot_above) - pre + v
        nq = jnp.sum(jnp.where(sfx >= float(_K), 1.0, 0.0), axis=0)
        cand = jnp.where(nq > 0.0, (b * _L).astype(jnp.float32) + nq - 1.0, -1.0)
        return tot_above + total, jnp.maximum(bstar_f, cand)
    _, bstar_f = lax.fori_loop(0, _BINV, sb, (jnp.float32(0.0),
                                              jnp.float32(-1.0)))
    th = bstar_f * (1.0 / _SCALE)

    def rb(i, carry):
        sa, ca = carry
        v = data_v[pl.ds(i * _L, _L)]
        m = v >= th
        return sa + jnp.where(m, v, 0.0), ca + jnp.where(m, 1.0, 0.0)
    sa, ca = lax.fori_loop(0, _VECS, rb,
                           (jnp.zeros((_L,), jnp.float32),
                            jnp.zeros((_L,), jnp.float32)))
    ssum = jnp.sum(sa, axis=0)
    scnt = jnp.sum(ca, axis=0)
    stage_v[...] = jnp.where(lane == 0, ssum, jnp.where(lane == 1, scnt, 0.0))
    pltpu.sync_copy(stage_v, slots.at[s])
    plsc.subcore_barrier()

    @pl.when(s == 0)
    def _():
        def ar(w, acc):
            pltpu.sync_copy(slots.at[w], stage_v)
            return acc + stage_v[...]
        acc = lax.fori_loop(0, _NS, ar, jnp.zeros((_L,), jnp.float32))
        stage_v[...] = acc
        pltpu.sync_copy(stage_v, out_hbm.at[c])


@functools.partial(jax.jit, static_argnames=())
def kernel(inputs, targets):
    x4 = inputs.reshape(_B, _C, _ROWS, _LANES)
    t3 = targets.astype(jnp.int32).reshape(_B, _ROWS, _LANES)
    nj = _ROWS // _RA
    ce = pl.pallas_call(
        _ce_kernel,
        grid=(_B, nj),
        in_specs=[
            pl.BlockSpec((1, _C, _RA, _LANES), lambda b, j: (b, 0, j, 0)),
            pl.BlockSpec((1, _RA, _LANES), lambda b, j: (b, j, 0)),
        ],
        out_specs=pl.BlockSpec((1, _RA, _LANES), lambda b, j: (b, j, 0)),
        out_shape=jax.ShapeDtypeStruct((_B, _ROWS, _LANES), jnp.float32),
        compiler_params=pltpu.CompilerParams(
            dimension_semantics=("arbitrary", "arbitrary"),
        ),
    )(x4, t3)
    cef = ce.reshape(_N)
    hist = _sc_hist(cef)
    parts = _sc_select(cef, hist)
    ssum = parts[0, 0] + parts[1, 0]
    scnt = parts[0, 1] + parts[1, 1]
    return ssum / scnt


# speedup vs baseline: 8.7144x; 1.0364x over previous
import functools

import jax
import jax.numpy as jnp
from jax import lax
from jax.experimental import pallas as pl
from jax.experimental.pallas import tpu as pltpu
from jax.experimental.pallas import tpu_sc as plsc

_B, _C, _H, _W = 4, 19, 512, 512
_HW = _H * _W
_LANES = 128
_ROWS = _HW // _LANES
_RA = 2048
_N = _B * _HW
_K = _N // 2
_NB = 4096
_SCALE = _NB / 16.0
_NC, _NS, _L = 2, 16, 16
_NW = _NC * _NS
_CHUNK = _N // _NW
_VECS = _CHUNK // _L
_BINV = _NB // _L
_BPW = _NB // _NS

_sc_mesh = plsc.VectorSubcoreMesh(core_axis_name="c", subcore_axis_name="s",
                                  num_cores=_NC, num_subcores=_NS)
_sc_params = pltpu.CompilerParams(needs_layout_passes=False)


def _ce_kernel(x_ref, t_ref, ce_ref):
    x = x_ref[0]
    t = t_ref[0]
    m = jnp.max(x, axis=0)
    s = jnp.sum(jnp.exp(x - m[None]), axis=0)
    lse = jnp.log(s) + m
    cls = jax.lax.broadcasted_iota(jnp.int32, (_C, _RA, _LANES), 0)
    xt = jnp.sum(jnp.where(cls == t[None], x, 0.0), axis=0)
    ce_ref[0] = lse - xt


@functools.partial(
    pl.kernel, mesh=_sc_mesh,
    out_type=jax.ShapeDtypeStruct((_NC, _NB), jnp.float32),
    scratch_types=[
        pltpu.VMEM((_CHUNK,), jnp.float32),
        pltpu.VMEM((_NB,), jnp.float32),
        pltpu.VMEM((_BPW,), jnp.float32),
        pltpu.VMEM((_NS, _BPW), jnp.float32),
        pltpu.VMEM_SHARED((_NS, _NB), jnp.float32),
    ],
    compiler_params=_sc_params,
)
def _sc_hist(ce_hbm, hist_hbm, data_v, hist_v, merge_v, mbuf_v, slots):
    c = lax.axis_index("c")
    s = lax.axis_index("s")
    wid = s * _NC + c
    zeros = jnp.zeros((_L,), jnp.float32)
    ones = jnp.ones((_L,), jnp.float32)

    pltpu.sync_copy(ce_hbm.at[pl.ds(wid * _CHUNK, _CHUNK)], data_v)

    def zb(i, _):
        hist_v[pl.ds(i * _L, _L)] = zeros
        return 0
    lax.fori_loop(0, _BINV, zb, 0)

    _UN = 8

    def hb(i, _):
        for u in range(_UN):
            v = data_v[pl.ds((i * _UN + u) * _L, _L)]
            idx = jnp.clip(v * _SCALE, 0.0, float(_NB - 1)).astype(jnp.int32)
            plsc.addupdate_scatter(hist_v, [idx], ones)
        return 0
    lax.fori_loop(0, _VECS // _UN, hb, 0)

    pltpu.sync_copy(hist_v, slots.at[s])
    plsc.subcore_barrier()

    pltpu.sync_copy(slots.at[:, pl.ds(s * _BPW, _BPW)], mbuf_v)

    def mr(i, _):
        sl = pl.ds(i * _L, _L)
        acc = mbuf_v[0, sl]
        for w in range(1, _NS):
            acc = acc + mbuf_v[w, sl]
        merge_v[sl] = acc
        return 0
    lax.fori_loop(0, _BPW // _L, mr, 0)

    pltpu.sync_copy(merge_v, hist_hbm.at[c, pl.ds(s * _BPW, _BPW)])


@functools.partial(
    pl.kernel, mesh=_sc_mesh,
    out_type=jax.ShapeDtypeStruct((_NC, _L), jnp.float32),
    scratch_types=[
        pltpu.VMEM((_CHUNK,), jnp.float32),
        pltpu.VMEM((_NC, _NB), jnp.float32),
        pltpu.VMEM((_L,), jnp.float32),
        pltpu.VMEM_SHARED((_NS, _L), jnp.float32),
    ],
    compiler_params=_sc_params,
)
def _sc_select(ce_hbm, hist_hbm, out_hbm, data_v, hist_v, stage_v, slots):
    c = lax.axis_index("c")
    s = lax.axis_index("s")
    wid = s * _NC + c
    lane = lax.iota(jnp.int32, _L)

    pltpu.sync_copy(ce_hbm.at[pl.ds(wid * _CHUNK, _CHUNK)], data_v)
    pltpu.sync_copy(hist_hbm, hist_v)

    def sb(j, carry):
        tot_above, bstar_f = carry
        b = _BINV - 1 - j
        v = hist_v[0, pl.ds(b * _L, _L)] + hist_v[1, pl.ds(b * _L, _L)]
        total = jnp.sum(v, axis=0)
        pre = plsc.cumsum(v)
        sfx = (total + tot_above) - pre + v
        nq = jnp.sum(jnp.where(sfx >= float(_K), 1.0, 0.0), axis=0)
        cand = jnp.where(nq > 0.0, (b * _L).astype(jnp.float32) + nq - 1.0, -1.0)
        return tot_above + total, jnp.maximum(bstar_f, cand)
    _, bstar_f = lax.fori_loop(0, _BINV, sb, (jnp.float32(0.0),
                                              jnp.float32(-1.0)))
    th = bstar_f * (1.0 / _SCALE)

    _UN = 8

    def rb(i, carry):
        sa, ca = carry
        for u in range(_UN):
            v = data_v[pl.ds((i * _UN + u) * _L, _L)]
            m = v >= th
            sa = sa + jnp.where(m, v, 0.0)
            ca = ca + jnp.where(m, 1.0, 0.0)
        return sa, ca
    sa, ca = lax.fori_loop(0, _VECS // _UN, rb,
                           (jnp.zeros((_L,), jnp.float32),
                            jnp.zeros((_L,), jnp.float32)))
    ssum = jnp.sum(sa, axis=0)
    scnt = jnp.sum(ca, axis=0)
    stage_v[...] = jnp.where(lane == 0, ssum, jnp.where(lane == 1, scnt, 0.0))
    pltpu.sync_copy(stage_v, slots.at[s])
    plsc.subcore_barrier()

    @pl.when(s == 0)
    def _():
        def ar(w, acc):
            pltpu.sync_copy(slots.at[w], stage_v)
            return acc + stage_v[...]
        acc = lax.fori_loop(0, _NS, ar, jnp.zeros((_L,), jnp.float32))
        stage_v[...] = acc
        pltpu.sync_copy(stage_v, out_hbm.at[c])


@functools.partial(jax.jit, static_argnames=())
def kernel(inputs, targets):
    x4 = inputs.reshape(_B, _C, _ROWS, _LANES)
    t3 = targets.astype(jnp.int32).reshape(_B, _ROWS, _LANES)
    nj = _ROWS // _RA
    ce = pl.pallas_call(
        _ce_kernel,
        grid=(_B, nj),
        in_specs=[
            pl.BlockSpec((1, _C, _RA, _LANES), lambda b, j: (b, 0, j, 0)),
            pl.BlockSpec((1, _RA, _LANES), lambda b, j: (b, j, 0)),
        ],
        out_specs=pl.BlockSpec((1, _RA, _LANES), lambda b, j: (b, j, 0)),
        out_shape=jax.ShapeDtypeStruct((_B, _ROWS, _LANES), jnp.float32),
        compiler_params=pltpu.CompilerParams(
            dimension_semantics=("arbitrary", "arbitrary"),
        ),
    )(x4, t3)
    cef = ce.reshape(_N)
    hist = _sc_hist(cef)
    parts = _sc_select(cef, hist)
    ssum = parts[0, 0] + parts[1, 0]
    scnt = parts[0, 1] + parts[1, 1]
    return ssum / scnt
